# baseline (speedup 1.0000x reference)
import functools

import jax
import jax.numpy as jnp
from jax import lax
from jax.experimental import pallas as pl
from jax.experimental.pallas import tpu as pltpu
from jax.experimental.pallas import tpu_sc as plsc

ROWS, COLS = 8192, 384

NSC_ROWS = 2048
NC, NS, L = 2, 16, 16
NW = NC * NS
ROWS_W = NSC_ROWS // NW
N_CHUNKS = 2
CHUNK_R = ROWS_W // N_CHUNKS
VPR = COLS // L
UNROLL = 4

TC_ROWS = ROWS - NSC_ROWS
TC_BR = 256
TC_GRID = TC_ROWS // TC_BR


def _sc_body(x_hbm, kmv_hbm, out_hbm, kmv_v, bufs, lsems, ssems):
    wid = lax.axis_index("s") * NC + lax.axis_index("c")
    base = wid * ROWS_W

    loads = []
    for c in range(N_CHUNKS):
        cp = pltpu.make_async_copy(
            x_hbm.at[pl.ds(base + c * CHUNK_R, CHUNK_R), :], bufs[c],
            lsems[c])
        cp.start()
        loads.append(cp)

    pltpu.sync_copy(kmv_hbm, kmv_v)
    snd, _ = plsc.sort_key_val(kmv_v[...], lax.iota(jnp.int32, 16))

    iota = lax.iota(jnp.int32, L)
    shifted = jnp.take_along_axis(snd, jnp.minimum(iota + 1, 15), axis=0)
    mv = 0.5 * (snd + shifted)
    mb7 = jnp.full((L,), mv[7], dtype=jnp.float32)
    mb3 = jnp.full((L,), mv[3], dtype=jnp.float32)
    mb11 = jnp.full((L,), mv[11], dtype=jnp.float32)
    c11 = jnp.full((L,), 11, dtype=jnp.int32)
    c3 = jnp.full((L,), 3, dtype=jnp.int32)
    p2 = jnp.full((L,), 2, dtype=jnp.int32)
    n2 = jnp.full((L,), -2, dtype=jnp.int32)
    p1 = jnp.full((L,), 1, dtype=jnp.int32)
    n1 = jnp.full((L,), -1, dtype=jnp.int32)
    i0 = jnp.zeros((L,), dtype=jnp.int32)

    def quant_vreg(xv):
        c8 = xv > mb7
        b4 = jnp.where(c8, mb11, mb3)
        c4 = xv > b4
        t = jnp.where(c8, c11, c3) + jnp.where(c4, p2, n2)
        b2 = jnp.take_along_axis(mv, t, axis=0)
        c2 = xv > b2
        t = t + jnp.where(c2, p1, n1)
        c1 = xv > jnp.take_along_axis(mv, t, axis=0)
        j = t + jnp.where(c1, p1, i0)
        return jnp.take_along_axis(snd, j, axis=0)

    stores = []
    for c in range(N_CHUNKS):
        loads[c].wait()
        buf = bufs[c]

        @plsc.parallel_loop(0, CHUNK_R, unroll=UNROLL)
        def quant_row(r):
            for v in range(VPR):
                sl = pl.ds(v * L, L)
                buf[r, sl] = quant_vreg(buf[r, sl])

        cp = pltpu.make_async_copy(
            buf, out_hbm.at[pl.ds(base + c * CHUNK_R, CHUNK_R), :],
            ssems[c])
        cp.start()
        stores.append(cp)

    for cp in stores:
        cp.wait()


def _sc_quantize(x2d, kmvalue):
    mesh = plsc.VectorSubcoreMesh(core_axis_name="c", subcore_axis_name="s")
    return pl.kernel(
        _sc_body,
        out_type=jax.ShapeDtypeStruct((ROWS, COLS), jnp.float32),
        mesh=mesh,
        scratch_types=[
            pltpu.VMEM((16,), jnp.float32),
            [pltpu.VMEM((CHUNK_R, COLS), jnp.float32)
             for _ in range(N_CHUNKS)],
            [pltpu.SemaphoreType.DMA for _ in range(N_CHUNKS)],
            [pltpu.SemaphoreType.DMA for _ in range(N_CHUNKS)],
        ],
        compiler_params=pltpu.CompilerParams(needs_layout_passes=False),
    )(x2d, kmvalue)


def _oem_pairs(n):
    pairs = []

    def merge(lo, hi, r):
        step = r * 2
        if step < hi - lo:
            merge(lo, hi, step)
            merge(lo + r, hi, step)
            for i in range(lo + r, hi - r, step):
                pairs.append((i, i + r))
        else:
            pairs.append((lo, lo + r))

    def sort(lo, hi):
        if hi - lo >= 2:
            mid = lo + (hi - lo) // 2
            sort(lo, mid)
            sort(mid, hi)
            merge(lo, hi, 1)

    sort(0, n)
    return pairs


_PAIRS16 = _oem_pairs(16)


def _tc_body(km_ref, x_ref, alias_ref, o_ref):
    del alias_ref
    v = [km_ref[i] for i in range(16)]
    for a, b in _PAIRS16:
        va, vb = v[a], v[b]
        v[a] = jnp.minimum(va, vb)
        v[b] = jnp.maximum(va, vb)
    m = [0.5 * (v[i] + v[i + 1]) for i in range(15)]

    xb = x_ref[...]
    r = jnp.full(xb.shape, v[0], dtype=jnp.float32)
    for k in range(15):
        r = jnp.where(xb > m[k], v[k + 1], r)
    o_ref[...] = r


def _tc_quantize(x2d, kmvalue, sc_out):
    return pl.pallas_call(
        _tc_body,
        out_shape=jax.ShapeDtypeStruct((ROWS, COLS), jnp.float32),
        grid=(TC_GRID,),
        in_specs=[
            pl.BlockSpec(memory_space=pltpu.SMEM),
            pl.BlockSpec((TC_BR, COLS),
                         lambda i: (i + NSC_ROWS // TC_BR, 0)),
            pl.BlockSpec(memory_space=pl.ANY),
        ],
        out_specs=pl.BlockSpec((TC_BR, COLS),
                               lambda i: (i + NSC_ROWS // TC_BR, 0)),
        input_output_aliases={2: 0},
    )(kmvalue, x2d, sc_out)


@jax.jit
def _quantize(x2d, kmvalue):
    sc_out = _sc_quantize(x2d, kmvalue)
    return _tc_quantize(x2d, kmvalue, sc_out)


def kernel(x, kmvalue):
    out = _quantize(x.reshape(ROWS, COLS), kmvalue)
    return out.reshape(x.shape)


# speedup vs baseline: 1.1165x; 1.1165x over previous
import functools

import jax
import jax.numpy as jnp
from jax import lax
from jax.experimental import pallas as pl
from jax.experimental.pallas import tpu as pltpu
from jax.experimental.pallas import tpu_sc as plsc

ROWS, COLS = 8192, 384

NSC_ROWS = 3072
NC, NS, L = 2, 16, 16
NW = NC * NS
ROWS_W = NSC_ROWS // NW
N_CHUNKS = 2
CHUNK_R = ROWS_W // N_CHUNKS
VPR = COLS // L
UNROLL = 4

TC_ROWS = ROWS - NSC_ROWS
TC_BR = 256
TC_GRID = TC_ROWS // TC_BR


def _sc_body(x_hbm, kmv_hbm, out_hbm, kmv_v, bufs, lsems, ssems):
    wid = lax.axis_index("s") * NC + lax.axis_index("c")
    base = wid * ROWS_W

    loads = []
    for c in range(N_CHUNKS):
        cp = pltpu.make_async_copy(
            x_hbm.at[pl.ds(base + c * CHUNK_R, CHUNK_R), :], bufs[c],
            lsems[c])
        cp.start()
        loads.append(cp)

    pltpu.sync_copy(kmv_hbm, kmv_v)
    snd, _ = plsc.sort_key_val(kmv_v[...], lax.iota(jnp.int32, 16))

    iota = lax.iota(jnp.int32, L)
    shifted = jnp.take_along_axis(snd, jnp.minimum(iota + 1, 15), axis=0)
    mv = 0.5 * (snd + shifted)
    mb7 = jnp.full((L,), mv[7], dtype=jnp.float32)
    mb3 = jnp.full((L,), mv[3], dtype=jnp.float32)
    mb11 = jnp.full((L,), mv[11], dtype=jnp.float32)
    c11 = jnp.full((L,), 11, dtype=jnp.int32)
    c3 = jnp.full((L,), 3, dtype=jnp.int32)
    p2 = jnp.full((L,), 2, dtype=jnp.int32)
    n2 = jnp.full((L,), -2, dtype=jnp.int32)
    p1 = jnp.full((L,), 1, dtype=jnp.int32)
    n1 = jnp.full((L,), -1, dtype=jnp.int32)
    i0 = jnp.zeros((L,), dtype=jnp.int32)

    def quant_vreg(xv):
        c8 = xv > mb7
        b4 = jnp.where(c8, mb11, mb3)
        c4 = xv > b4
        t = jnp.where(c8, c11, c3) + jnp.where(c4, p2, n2)
        b2 = jnp.take_along_axis(mv, t, axis=0)
        c2 = xv > b2
        t = t + jnp.where(c2, p1, n1)
        c1 = xv > jnp.take_along_axis(mv, t, axis=0)
        j = t + jnp.where(c1, p1, i0)
        return jnp.take_along_axis(snd, j, axis=0)

    stores = []
    for c in range(N_CHUNKS):
        loads[c].wait()
        buf = bufs[c]

        @plsc.parallel_loop(0, CHUNK_R, unroll=UNROLL)
        def quant_row(r):
            for v in range(VPR):
                sl = pl.ds(v * L, L)
                buf[r, sl] = quant_vreg(buf[r, sl])

        cp = pltpu.make_async_copy(
            buf, out_hbm.at[pl.ds(base + c * CHUNK_R, CHUNK_R), :],
            ssems[c])
        cp.start()
        stores.append(cp)

    for cp in stores:
        cp.wait()


def _sc_quantize(x2d, kmvalue):
    mesh = plsc.VectorSubcoreMesh(core_axis_name="c", subcore_axis_name="s")
    return pl.kernel(
        _sc_body,
        out_type=jax.ShapeDtypeStruct((NSC_ROWS, COLS), jnp.float32),
        mesh=mesh,
        scratch_types=[
            pltpu.VMEM((16,), jnp.float32),
            [pltpu.VMEM((CHUNK_R, COLS), jnp.float32)
             for _ in range(N_CHUNKS)],
            [pltpu.SemaphoreType.DMA for _ in range(N_CHUNKS)],
            [pltpu.SemaphoreType.DMA for _ in range(N_CHUNKS)],
        ],
        compiler_params=pltpu.CompilerParams(needs_layout_passes=False),
    )(x2d, kmvalue)


def _oem_pairs(n):
    pairs = []

    def merge(lo, hi, r):
        step = r * 2
        if step < hi - lo:
            merge(lo, hi, step)
            merge(lo + r, hi, step)
            for i in range(lo + r, hi - r, step):
                pairs.append((i, i + r))
        else:
            pairs.append((lo, lo + r))

    def sort(lo, hi):
        if hi - lo >= 2:
            mid = lo + (hi - lo) // 2
            sort(lo, mid)
            sort(mid, hi)
            merge(lo, hi, 1)

    sort(0, n)
    return pairs


_PAIRS16 = _oem_pairs(16)


def _tc_body(km_ref, x_ref, o_ref):
    v = [km_ref[i] for i in range(16)]
    for a, b in _PAIRS16:
        va, vb = v[a], v[b]
        v[a] = jnp.minimum(va, vb)
        v[b] = jnp.maximum(va, vb)
    m = [0.5 * (v[i] + v[i + 1]) for i in range(15)]

    xb = x_ref[...]
    r = jnp.full(xb.shape, v[0], dtype=jnp.float32)
    for k in range(15):
        r = jnp.where(xb > m[k], v[k + 1], r)
    o_ref[...] = r


def _tc_quantize(x2d, kmvalue):
    return pl.pallas_call(
        _tc_body,
        out_shape=jax.ShapeDtypeStruct((TC_ROWS, COLS), jnp.float32),
        grid=(TC_GRID,),
        in_specs=[
            pl.BlockSpec(memory_space=pltpu.SMEM),
            pl.BlockSpec((TC_BR, COLS),
                         lambda i: (i + NSC_ROWS // TC_BR, 0)),
        ],
        out_specs=pl.BlockSpec((TC_BR, COLS), lambda i: (i, 0)),
    )(kmvalue, x2d)


@jax.jit
def _quantize(x2d, kmvalue):
    sc_out = _sc_quantize(x2d, kmvalue)
    tc_out = _tc_quantize(x2d, kmvalue)
    return jnp.concatenate([sc_out, tc_out], axis=0)


def kernel(x, kmvalue):
    out = _quantize(x.reshape(ROWS, COLS), kmvalue)
    return out.reshape(x.shape)
